# baseline (speedup 1.0000x reference)
import functools

import jax
import jax.numpy as jnp
from jax import lax
from jax.experimental import pallas as pl
from jax.experimental.pallas import tpu as pltpu
from jax.experimental.pallas import tpu_sc as plsc

C = 128
K = 32
M = 512
NS = 2
GW = 128
EPS = 1e-5
INV_SCALE = 1.0 / 30.0


BF = jnp.bfloat16


def _gelu(x):
    c = jnp.asarray(0.7071067811865476, x.dtype)
    h = jnp.asarray(0.5, x.dtype)
    return x * (h * lax.erf(x * c) + h)


def _ln(x, g, b):
    m = jnp.mean(x, axis=-1, keepdims=True)
    d = x - m
    v = jnp.mean(d * d, axis=-1, keepdims=True)
    return d * lax.rsqrt(v + EPS) * g + b


def _dot(a, b):
    return jnp.dot(a.astype(BF), b.astype(BF),
                   preferred_element_type=jnp.float32)


_TC_PARAMS = pltpu.CompilerParams(dimension_semantics=("parallel",))


def _stage1_body(hv_ref, he_ref, gt_ref, man_ref, mv_ref,
                 w1v_ref, w1e_ref, w1g_ref, b1_ref, w2_ref, b2_ref,
                 w3_ref, b3_ref, win_ref, bin_ref, wout_ref, bout_ref,
                 ln1g_ref, ln1b_ref, ln2g_ref, ln2b_ref,
                 w11v_ref, b11_ref, w11g_ref,
                 hv_out_ref, p2_out_ref, v2_out_ref):
    hv = hv_ref[...]
    t = (_dot(he_ref[...].reshape(M * K, C), w1e_ref[...])
         + _dot(gt_ref[...], w1g_ref[...]))
    v = _dot(hv, w1v_ref[...]) + b1_ref[...]
    t = (t.reshape(M, K, C) + v[:, None, :]).reshape(M * K, C)
    h = _gelu(t.astype(BF))
    h = _gelu((_dot(h, w2_ref[...]) + b2_ref[...]).astype(BF))
    man = man_ref[...]
    h3 = h.reshape(M, K, C).astype(jnp.float32) * man[:, :, None]
    z = jnp.sum(h3, axis=1)
    cnt = jnp.sum(man, axis=1, keepdims=True)
    dh = (_dot(z, w3_ref[...]) + cnt * b3_ref[...]) * INV_SCALE
    x = _ln(hv + dh, ln1g_ref[...], ln1b_ref[...])
    f = _dot(_gelu((_dot(x, win_ref[...]) + bin_ref[...]).astype(BF)),
             wout_ref[...])
    x = _ln(x + f + bout_ref[...], ln2g_ref[...], ln2b_ref[...])
    x = x * mv_ref[...]
    hv_out_ref[...] = x
    p2_out_ref[...] = _dot(x, w11g_ref[...])
    v2_out_ref[...] = _dot(x, w11v_ref[...]) + b11_ref[...]


def _stage2_body(he_ref, gt_ref, v2_ref, w11e_ref, w12_ref, b12_ref,
                 w13_ref, b13_ref, ln3g_ref, ln3b_ref, *refs):
    out_ref = refs[-1]
    he = he_ref[...].reshape(M * K, C)
    t = _dot(he, w11e_ref[...]) + gt_ref[...]
    t = (t.reshape(M, K, C) + v2_ref[...][:, None, :]).reshape(M * K, C)
    h = _gelu(t.astype(BF))
    h = _gelu((_dot(h, w12_ref[...]) + b12_ref[...]).astype(BF))
    hm = _dot(h, w13_ref[...]) + b13_ref[...]
    out = _ln(he + hm, ln3g_ref[...], ln3b_ref[...])
    out_ref[...] = out.reshape(1, M, K, C)


def _full(shape):
    nd = len(shape)
    return pl.BlockSpec(shape, lambda i: (0,) * nd)


def _sc_gather(table, idx2):
    n_idx = idx2.shape[1]
    mesh = plsc.VectorSubcoreMesh(core_axis_name="core",
                                  subcore_axis_name="subcore")

    @functools.partial(
        pl.kernel,
        out_type=jax.ShapeDtypeStruct((n_idx, table.shape[1]), table.dtype),
        mesh=mesh)
    def k(x_hbm, i_hbm, o_hbm):
        def body(i_vmem, o_vmem):
            pltpu.sync_copy(x_hbm.at[i_vmem.at[0]], o_vmem)

        pltpu.emit_pipeline(
            body,
            grid=(n_idx // GW,),
            in_specs=[pl.BlockSpec((1, GW), lambda i: (0, i))],
            out_specs=[pl.BlockSpec((GW, table.shape[1]),
                                    lambda i: (i, 0))],
            core_axis_name=("core", "subcore"),
            dimension_semantics=(pltpu.PARALLEL,),
        )(i_hbm, o_hbm)

    return k(table, idx2)


def kernel(h_V, h_E, E_idx, mask_V, mask_attend, W1, b1, W2, b2, W3, b3,
           W11, b11, W12, b12, W13, b13, W_in, b_in, W_out, b_out,
           g1, be1, g2, be2, g3, be3):
    B, N, _ = h_V.shape
    R = B * N
    E = R * K
    T = R // M

    NBN = N // M
    TH = T // NS
    RH = R // NS
    EH = E // NS

    hv2 = h_V.reshape(R, C)
    offs = (jnp.arange(B, dtype=jnp.int32) * N)[:, None, None]
    idx2 = (E_idx.astype(jnp.int32) + offs).reshape(1, E)
    man = mask_attend.reshape(R, K)
    mv = mask_V.reshape(R, 1)

    W1v, W1e, W1g = W1[:C], W1[C:2 * C], W1[2 * C:]
    W11v, W11e, W11g = W11[:C], W11[C:2 * C], W11[2 * C:]
    r2 = lambda a: a.reshape(1, -1)

    G1h = [_sc_gather(hv2, lax.slice(idx2, (0, h * EH), (1, (h + 1) * EH)))
           for h in range(NS)]

    w_specs = [
        _full((C, C)), _full((C, C)), _full((C, C)), _full((1, C)),
        _full((C, C)), _full((1, C)), _full((C, C)), _full((1, C)),
        _full((C, 4 * C)), _full((1, 4 * C)), _full((4 * C, C)),
        _full((1, C)),
        _full((1, C)), _full((1, C)), _full((1, C)), _full((1, C)),
        _full((C, C)), _full((1, C)), _full((C, C)),
    ]
    w_args = (W1v, W1e, W1g, r2(b1), W2, r2(b2), W3, r2(b3),
              W_in, r2(b_in), W_out, r2(b_out),
              r2(g1), r2(be1), r2(g2), r2(be2),
              W11v, r2(b11), W11g)

    parts = []
    for h in range(NS):
        off = h * TH
        node_off = pl.BlockSpec((M, C), lambda i, off=off: (i + off, 0))
        he_spec = pl.BlockSpec(
            (1, M, K, C),
            lambda i, off=off: ((i + off) // NBN, (i + off) % NBN, 0, 0))
        parts.append(pl.pallas_call(
            _stage1_body,
            grid=(TH,),
            in_specs=[
                node_off, he_spec,
                pl.BlockSpec((M * K, C), lambda i: (i, 0)),
                pl.BlockSpec((M, K), lambda i, off=off: (i + off, 0)),
                pl.BlockSpec((M, 1), lambda i, off=off: (i + off, 0)),
            ] + w_specs,
            out_specs=[pl.BlockSpec((M, C), lambda i: (i, 0))] * 3,
            out_shape=[jax.ShapeDtypeStruct((RH, C), jnp.float32)] * 3,
            compiler_params=_TC_PARAMS,
        )(hv2, h_E, G1h[h], man, mv, *w_args))

    hv_new = jnp.concatenate([p[0] for p in parts], axis=0)
    P2 = jnp.concatenate([p[1] for p in parts], axis=0)
    V2 = jnp.concatenate([p[2] for p in parts], axis=0)

    G2h = [_sc_gather(P2, lax.slice(idx2, (0, h * EH), (1, (h + 1) * EH)))
           for h in range(NS)]

    he_out = None
    for h in range(NS):
        off = h * TH
        he4_spec = pl.BlockSpec(
            (1, M, K, C),
            lambda i, off=off: ((i + off) // NBN, (i + off) % NBN, 0, 0))
        in_specs = [
            he4_spec,
            pl.BlockSpec((M * K, C), lambda i: (i, 0)),
            pl.BlockSpec((M, C), lambda i, off=off: (i + off, 0)),
            _full((C, C)), _full((C, C)), _full((1, C)),
            _full((C, C)), _full((1, C)), _full((1, C)), _full((1, C)),
        ]
        args = [h_E, G2h[h], V2, W11e, W12, r2(b12), W13, r2(b13),
                r2(g3), r2(be3)]
        aliases = {}
        if h > 0:
            in_specs.append(pl.BlockSpec(memory_space=pltpu.MemorySpace.HBM))
            args.append(he_out)
            aliases = {len(args) - 1: 0}
        he_out = pl.pallas_call(
            _stage2_body,
            grid=(TH,),
            in_specs=in_specs,
            out_specs=he4_spec,
            out_shape=jax.ShapeDtypeStruct((B, N, K, C), jnp.float32),
            input_output_aliases=aliases,
            compiler_params=_TC_PARAMS,
        )(*args)

    return hv_new.reshape(B, N, C), he_out


# speedup vs baseline: 1.0046x; 1.0046x over previous
import functools

import jax
import jax.numpy as jnp
from jax import lax
from jax.experimental import pallas as pl
from jax.experimental.pallas import tpu as pltpu
from jax.experimental.pallas import tpu_sc as plsc

C = 128
K = 32
M = 256
NS = 4
GW = 128
EPS = 1e-5
INV_SCALE = 1.0 / 30.0


BF = jnp.bfloat16


def _gelu(x):
    c = jnp.asarray(0.7071067811865476, x.dtype)
    h = jnp.asarray(0.5, x.dtype)
    return x * (h * lax.erf(x * c) + h)


def _ln(x, g, b):
    m = jnp.mean(x, axis=-1, keepdims=True)
    d = x - m
    v = jnp.mean(d * d, axis=-1, keepdims=True)
    return d * lax.rsqrt(v + EPS) * g + b


def _dot(a, b):
    return jnp.dot(a.astype(BF), b.astype(BF),
                   preferred_element_type=jnp.float32)


_TC_PARAMS = pltpu.CompilerParams(dimension_semantics=("parallel",))


def _stage1_body(hv_ref, he_ref, gt_ref, man_ref, mv_ref,
                 w1v_ref, w1e_ref, w1g_ref, b1_ref, w2_ref, b2_ref,
                 w3_ref, b3_ref, win_ref, bin_ref, wout_ref, bout_ref,
                 ln1g_ref, ln1b_ref, ln2g_ref, ln2b_ref,
                 w11v_ref, b11_ref, w11g_ref,
                 hv_out_ref, p2_out_ref, v2_out_ref):
    hv = hv_ref[...]
    t = (_dot(he_ref[...].reshape(M * K, C), w1e_ref[...])
         + _dot(gt_ref[...], w1g_ref[...]))
    v = _dot(hv, w1v_ref[...]) + b1_ref[...]
    t = (t.reshape(M, K, C) + v[:, None, :]).reshape(M * K, C)
    h = _gelu(t.astype(BF))
    h = _gelu((_dot(h, w2_ref[...]) + b2_ref[...]).astype(BF))
    man = man_ref[...]
    h3 = h.reshape(M, K, C).astype(jnp.float32) * man[:, :, None]
    z = jnp.sum(h3, axis=1)
    cnt = jnp.sum(man, axis=1, keepdims=True)
    dh = (_dot(z, w3_ref[...]) + cnt * b3_ref[...]) * INV_SCALE
    x = _ln(hv + dh, ln1g_ref[...], ln1b_ref[...])
    f = _dot(_gelu((_dot(x, win_ref[...]) + bin_ref[...]).astype(BF)),
             wout_ref[...])
    x = _ln(x + f + bout_ref[...], ln2g_ref[...], ln2b_ref[...])
    x = x * mv_ref[...]
    hv_out_ref[...] = x
    p2_out_ref[...] = _dot(x, w11g_ref[...])
    v2_out_ref[...] = _dot(x, w11v_ref[...]) + b11_ref[...]


def _stage2_body(he_ref, gt_ref, v2_ref, w11e_ref, w12_ref, b12_ref,
                 w13_ref, b13_ref, ln3g_ref, ln3b_ref, *refs):
    out_ref = refs[-1]
    he = he_ref[...].reshape(M * K, C)
    t = _dot(he, w11e_ref[...]) + gt_ref[...]
    t = (t.reshape(M, K, C) + v2_ref[...][:, None, :]).reshape(M * K, C)
    h = _gelu(t.astype(BF))
    h = _gelu((_dot(h, w12_ref[...]) + b12_ref[...]).astype(BF))
    hm = _dot(h, w13_ref[...]) + b13_ref[...]
    out = _ln(he + hm, ln3g_ref[...], ln3b_ref[...])
    out_ref[...] = out.reshape(1, M, K, C)


def _full(shape):
    nd = len(shape)
    return pl.BlockSpec(shape, lambda i: (0,) * nd)


def _sc_gather(table, idx2):
    n_idx = idx2.shape[1]
    mesh = plsc.VectorSubcoreMesh(core_axis_name="core",
                                  subcore_axis_name="subcore")

    @functools.partial(
        pl.kernel,
        out_type=jax.ShapeDtypeStruct((n_idx, table.shape[1]), table.dtype),
        mesh=mesh)
    def k(x_hbm, i_hbm, o_hbm):
        def body(i_vmem, o_vmem):
            pltpu.sync_copy(x_hbm.at[i_vmem.at[0]], o_vmem)

        pltpu.emit_pipeline(
            body,
            grid=(n_idx // GW,),
            in_specs=[pl.BlockSpec((1, GW), lambda i: (0, i))],
            out_specs=[pl.BlockSpec((GW, table.shape[1]),
                                    lambda i: (i, 0))],
            core_axis_name=("core", "subcore"),
            dimension_semantics=(pltpu.PARALLEL,),
        )(i_hbm, o_hbm)

    return k(table, idx2)


def kernel(h_V, h_E, E_idx, mask_V, mask_attend, W1, b1, W2, b2, W3, b3,
           W11, b11, W12, b12, W13, b13, W_in, b_in, W_out, b_out,
           g1, be1, g2, be2, g3, be3):
    B, N, _ = h_V.shape
    R = B * N
    E = R * K
    T = R // M

    NBN = N // M
    TH = T // NS
    RH = R // NS
    EH = E // NS

    hv2 = h_V.reshape(R, C)
    offs = (jnp.arange(B, dtype=jnp.int32) * N)[:, None, None]
    idx2 = (E_idx.astype(jnp.int32) + offs).reshape(1, E)
    man = mask_attend.reshape(R, K)
    mv = mask_V.reshape(R, 1)

    W1v, W1e, W1g = W1[:C], W1[C:2 * C], W1[2 * C:]
    W11v, W11e, W11g = W11[:C], W11[C:2 * C], W11[2 * C:]
    r2 = lambda a: a.reshape(1, -1)

    G1h = [_sc_gather(hv2, lax.slice(idx2, (0, h * EH), (1, (h + 1) * EH)))
           for h in range(NS)]

    w_specs = [
        _full((C, C)), _full((C, C)), _full((C, C)), _full((1, C)),
        _full((C, C)), _full((1, C)), _full((C, C)), _full((1, C)),
        _full((C, 4 * C)), _full((1, 4 * C)), _full((4 * C, C)),
        _full((1, C)),
        _full((1, C)), _full((1, C)), _full((1, C)), _full((1, C)),
        _full((C, C)), _full((1, C)), _full((C, C)),
    ]
    w_args = (W1v, W1e, W1g, r2(b1), W2, r2(b2), W3, r2(b3),
              W_in, r2(b_in), W_out, r2(b_out),
              r2(g1), r2(be1), r2(g2), r2(be2),
              W11v, r2(b11), W11g)

    parts = []
    for h in range(NS):
        off = h * TH
        node_off = pl.BlockSpec((M, C), lambda i, off=off: (i + off, 0))
        he_spec = pl.BlockSpec(
            (1, M, K, C),
            lambda i, off=off: ((i + off) // NBN, (i + off) % NBN, 0, 0))
        parts.append(pl.pallas_call(
            _stage1_body,
            grid=(TH,),
            in_specs=[
                node_off, he_spec,
                pl.BlockSpec((M * K, C), lambda i: (i, 0)),
                pl.BlockSpec((M, K), lambda i, off=off: (i + off, 0)),
                pl.BlockSpec((M, 1), lambda i, off=off: (i + off, 0)),
            ] + w_specs,
            out_specs=[pl.BlockSpec((M, C), lambda i: (i, 0))] * 3,
            out_shape=[jax.ShapeDtypeStruct((RH, C), jnp.float32)] * 3,
            compiler_params=_TC_PARAMS,
        )(hv2, h_E, G1h[h], man, mv, *w_args))

    hv_new = jnp.concatenate([p[0] for p in parts], axis=0)
    P2 = jnp.concatenate([p[1] for p in parts], axis=0)
    V2 = jnp.concatenate([p[2] for p in parts], axis=0)

    G2h = [_sc_gather(P2, lax.slice(idx2, (0, h * EH), (1, (h + 1) * EH)))
           for h in range(NS)]

    he_out = None
    for h in range(NS):
        off = h * TH
        he4_spec = pl.BlockSpec(
            (1, M, K, C),
            lambda i, off=off: ((i + off) // NBN, (i + off) % NBN, 0, 0))
        in_specs = [
            he4_spec,
            pl.BlockSpec((M * K, C), lambda i: (i, 0)),
            pl.BlockSpec((M, C), lambda i, off=off: (i + off, 0)),
            _full((C, C)), _full((C, C)), _full((1, C)),
            _full((C, C)), _full((1, C)), _full((1, C)), _full((1, C)),
        ]
        args = [h_E, G2h[h], V2, W11e, W12, r2(b12), W13, r2(b13),
                r2(g3), r2(be3)]
        aliases = {}
        if h > 0:
            in_specs.append(pl.BlockSpec(memory_space=pltpu.MemorySpace.HBM))
            args.append(he_out)
            aliases = {len(args) - 1: 0}
        he_out = pl.pallas_call(
            _stage2_body,
            grid=(TH,),
            in_specs=in_specs,
            out_specs=he4_spec,
            out_shape=jax.ShapeDtypeStruct((B, N, K, C), jnp.float32),
            input_output_aliases=aliases,
            compiler_params=_TC_PARAMS,
        )(*args)

    return hv_new.reshape(B, N, C), he_out
